# baseline (speedup 1.0000x reference)
import functools

import jax
import jax.numpy as jnp
from jax.experimental import pallas as pl
from jax.experimental.pallas import tpu as pltpu


def _body(x_ref, ws_ref, b_ref, o_ref, *, H, C):
    f32, bf = jnp.float32, jnp.bfloat16
    WC = x_ref.shape[1]

    r = jax.lax.broadcasted_iota(jnp.int32, (WC, C), 0)
    c = jax.lax.broadcasted_iota(jnp.int32, (WC, C), 1)
    p1 = (r % C == c).astype(bf)
    p2 = p1.T
    xo_blk = jax.lax.broadcasted_iota(jnp.int32, (WC, WC), 0) // C
    xi_blk = jax.lax.broadcasted_iota(jnp.int32, (WC, WC), 1) // C
    off = xi_blk - xo_blk + 1
    wbt = []
    for band in range(12):
        acc = jnp.zeros((WC, WC), f32)
        for kx in range(3):
            m = ws_ref[band, kx].astype(bf)
            mp = jnp.dot(m, p2, preferred_element_type=f32)
            t = jnp.dot(p1, mp.astype(bf), preferred_element_type=f32)
            acc += jnp.where(off == kx, t, 0.0)
        wbt.append(acc.astype(bf))

    def conv_bn(a, i):
        outs = []
        for h in range(H):
            y = jnp.dot(wbt[3 * i + 1], a[h], preferred_element_type=f32)
            if h > 0:
                y += jnp.dot(wbt[3 * i], a[h - 1], preferred_element_type=f32)
            if h < H - 1:
                y += jnp.dot(wbt[3 * i + 2], a[h + 1],
                             preferred_element_type=f32)
            outs.append(y + b_ref[i])
        return outs

    x0 = [x_ref[h] for h in range(H)]
    h1 = [jnp.maximum(y, 0.0).astype(bf) for y in conv_bn(x0, 0)]
    x1 = [jnp.maximum(x0[h].astype(f32) + y, 0.0)
          for h, y in enumerate(conv_bn(h1, 1))]
    x1b = [v.astype(bf) for v in x1]
    h2 = [jnp.maximum(y, 0.0).astype(bf) for y in conv_bn(x1b, 2)]
    for h, y in enumerate(conv_bn(h2, 3)):
        o_ref[h] = jnp.maximum(x1[h] + y, 0.0)


def kernel(x, b1_w1, b1_scale1, b1_bias1, b1_w2, b1_scale2, b1_bias2,
           b2_w1, b2_scale1, b2_bias1, b2_w2, b2_scale2, b2_bias2):
    N, H, W, C = x.shape
    WC = W * C
    NSPLIT = 2 if N % 256 == 0 else 1
    NL = N // NSPLIT

    ws = jnp.stack([
        b1_w1 * b1_scale1, b1_w2 * b1_scale2,
        b2_w1 * b2_scale1, b2_w2 * b2_scale2,
    ]).reshape(12, 3, C, C).swapaxes(-1, -2)
    bias = jnp.stack([
        jnp.tile(b1_bias1, W), jnp.tile(b1_bias2, W),
        jnp.tile(b2_bias1, W), jnp.tile(b2_bias2, W),
    ]).reshape(4, WC, 1).astype(jnp.float32)

    xt = x.transpose(1, 2, 3, 0).reshape(H, WC, N).astype(jnp.bfloat16)

    out = pl.pallas_call(
        functools.partial(_body, H=H, C=C),
        out_shape=jax.ShapeDtypeStruct((H, WC, N), jnp.float32),
        grid=(NSPLIT,),
        in_specs=[
            pl.BlockSpec((H, WC, NL), lambda j: (0, 0, j)),
            pl.BlockSpec((12, 3, C, C), lambda j: (0, 0, 0, 0)),
            pl.BlockSpec((4, WC, 1), lambda j: (0, 0, 0)),
        ],
        out_specs=pl.BlockSpec((H, WC, NL), lambda j: (0, 0, j)),
        compiler_params=pltpu.CompilerParams(
            dimension_semantics=("parallel",),
            vmem_limit_bytes=60 * 1024 * 1024,
        ),
    )(xt, ws, bias)
    return out.reshape(H, W, C, N).transpose(3, 0, 1, 2)


# speedup vs baseline: 1.4621x; 1.4621x over previous
import functools

import jax
import jax.numpy as jnp
from jax.experimental import pallas as pl
from jax.experimental.pallas import tpu as pltpu

_PB = 4


def _windows(W, C):
    WC = W * C
    nblk = max(W // _PB, 1)
    win = min(_PB + 2, W) * C
    starts = [min(max(C * (_PB * w0 - 1), 0), WC - win) for w0 in range(nblk)]
    return nblk, win, starts


def _body(x_ref, wbt_ref, b_ref, o_ref, *, H, C):
    f32, bf = jnp.float32, jnp.bfloat16
    WC = x_ref.shape[1]
    W = WC // C
    nblk, win, starts = _windows(W, C)
    pbc = _PB * C

    def conv_bn(a, i):
        outs = []
        for h in range(H):
            blocks = []
            for w0 in range(nblk):
                s = starts[w0]
                y = jnp.dot(wbt_ref[3 * i + 1, w0], a[h][s:s + win],
                            preferred_element_type=f32)
                if h > 0:
                    y += jnp.dot(wbt_ref[3 * i, w0], a[h - 1][s:s + win],
                                 preferred_element_type=f32)
                if h < H - 1:
                    y += jnp.dot(wbt_ref[3 * i + 2, w0], a[h + 1][s:s + win],
                                 preferred_element_type=f32)
                blocks.append(y + b_ref[i, pbc * w0:pbc * (w0 + 1)])
            outs.append(jnp.concatenate(blocks, axis=0)
                        if nblk > 1 else blocks[0])
        return outs

    x0 = [x_ref[h] for h in range(H)]
    h1 = [jnp.maximum(y, 0.0).astype(bf) for y in conv_bn(x0, 0)]
    x1 = [jnp.maximum(x0[h].astype(f32) + y, 0.0)
          for h, y in enumerate(conv_bn(h1, 1))]
    x1b = [v.astype(bf) for v in x1]
    h2 = [jnp.maximum(y, 0.0).astype(bf) for y in conv_bn(x1b, 2)]
    for h, y in enumerate(conv_bn(h2, 3)):
        o_ref[h] = jnp.maximum(x1[h] + y, 0.0)


def _prep_body(ws_ref, mask_ref, wb_ref, *, C):
    WC = mask_ref.shape[1]
    W = WC // C
    nblk, win, starts = _windows(W, C)
    pbc = _PB * C
    r = jax.lax.broadcasted_iota(jnp.int32, (WC, C), 0)
    c = jax.lax.broadcasted_iota(jnp.int32, (WC, C), 1)
    p1 = (r % C == c).astype(jnp.bfloat16)
    p2 = p1.T
    acc = jnp.zeros((WC, WC), jnp.float32)
    for kx in range(3):
        m = ws_ref[0, kx].astype(jnp.bfloat16)
        mp = jnp.dot(m, p2, preferred_element_type=jnp.float32)
        t = jnp.dot(p1, mp.astype(jnp.bfloat16),
                    preferred_element_type=jnp.float32)
        acc += t * mask_ref[kx]
    band = acc.astype(jnp.bfloat16)
    for w0 in range(nblk):
        s = starts[w0]
        wb_ref[0, w0] = band[pbc * w0:pbc * (w0 + 1), s:s + win]


def kernel(x, b1_w1, b1_scale1, b1_bias1, b1_w2, b1_scale2, b1_bias2,
           b2_w1, b2_scale1, b2_bias1, b2_w2, b2_scale2, b2_bias2):
    N, H, W, C = x.shape
    WC = W * C
    NSPLIT = 2 if N % 256 == 0 else 1
    NL = N // NSPLIT
    nblk, win, _ = _windows(W, C)

    ws = jnp.stack([
        b1_w1 * b1_scale1, b1_w2 * b1_scale2,
        b2_w1 * b2_scale1, b2_w2 * b2_scale2,
    ]).reshape(12, 3, C, C).swapaxes(-1, -2)
    xo_blk = jax.lax.broadcasted_iota(jnp.int32, (3, WC, WC), 1) // C
    xi_blk = jax.lax.broadcasted_iota(jnp.int32, (3, WC, WC), 2) // C
    kx_i = jax.lax.broadcasted_iota(jnp.int32, (3, WC, WC), 0)
    masks = (xi_blk - xo_blk + 1 == kx_i).astype(jnp.float32)
    wbt = pl.pallas_call(
        functools.partial(_prep_body, C=C),
        out_shape=jax.ShapeDtypeStruct((12, nblk, _PB * C, win), jnp.bfloat16),
        grid=(12,),
        in_specs=[
            pl.BlockSpec((1, 3, C, C), lambda i: (i, 0, 0, 0)),
            pl.BlockSpec((3, WC, WC), lambda i: (0, 0, 0)),
        ],
        out_specs=pl.BlockSpec((1, nblk, _PB * C, win), lambda i: (i, 0, 0, 0)),
        compiler_params=pltpu.CompilerParams(
            dimension_semantics=("parallel",),
        ),
    )(ws, masks)

    bias = jnp.stack([
        jnp.tile(b1_bias1, W), jnp.tile(b1_bias2, W),
        jnp.tile(b2_bias1, W), jnp.tile(b2_bias2, W),
    ]).reshape(4, WC, 1).astype(jnp.float32)

    xt = x.transpose(1, 2, 3, 0).reshape(H, WC, N).astype(jnp.bfloat16)

    out = pl.pallas_call(
        functools.partial(_body, H=H, C=C),
        out_shape=jax.ShapeDtypeStruct((H, WC, N), jnp.float32),
        grid=(NSPLIT,),
        in_specs=[
            pl.BlockSpec((H, WC, NL), lambda j: (0, 0, j)),
            pl.BlockSpec((12, nblk, _PB * C, win), lambda j: (0, 0, 0, 0)),
            pl.BlockSpec((4, WC, 1), lambda j: (0, 0, 0)),
        ],
        out_specs=pl.BlockSpec((H, WC, NL), lambda j: (0, 0, j)),
        compiler_params=pltpu.CompilerParams(
            dimension_semantics=("parallel",),
            vmem_limit_bytes=56 * 1024 * 1024,
        ),
    )(xt, wbt, bias)
    return out.reshape(H, W, C, N).transpose(3, 0, 1, 2)


# speedup vs baseline: 1.8051x; 1.2346x over previous
import functools

import jax
import jax.numpy as jnp
from jax.experimental import pallas as pl
from jax.experimental.pallas import tpu as pltpu

_PB = 4


def _windows(W, C):
    WC = W * C
    nblk = max(W // _PB, 1)
    win = min(_PB + 2, W) * C
    starts = [min(max(C * (_PB * w0 - 1), 0), WC - win) for w0 in range(nblk)]
    return nblk, win, starts


def _body(x_ref, wbt_ref, b_ref, o_ref, *, H, C):
    f32, bf = jnp.float32, jnp.bfloat16
    WC = x_ref.shape[1]
    W = WC // C
    nblk, win, starts = _windows(W, C)
    pbc = _PB * C

    def conv_bn(a, i):
        outs = []
        for h in range(H):
            blocks = []
            for w0 in range(nblk):
                s = starts[w0]
                y = jnp.dot(wbt_ref[3 * i + 1, w0], a[h][s:s + win],
                            preferred_element_type=f32)
                if h > 0:
                    y += jnp.dot(wbt_ref[3 * i, w0], a[h - 1][s:s + win],
                                 preferred_element_type=f32)
                if h < H - 1:
                    y += jnp.dot(wbt_ref[3 * i + 2, w0], a[h + 1][s:s + win],
                                 preferred_element_type=f32)
                blocks.append(y + b_ref[i, pbc * w0:pbc * (w0 + 1)])
            outs.append(jnp.concatenate(blocks, axis=0)
                        if nblk > 1 else blocks[0])
        return outs

    x0 = [x_ref[h] for h in range(H)]
    x0b = [v.astype(bf) for v in x0]
    h1 = [jnp.maximum(y, 0.0).astype(bf) for y in conv_bn(x0b, 0)]
    x1 = [jnp.maximum(x0[h] + y, 0.0)
          for h, y in enumerate(conv_bn(h1, 1))]
    x1b = [v.astype(bf) for v in x1]
    h2 = [jnp.maximum(y, 0.0).astype(bf) for y in conv_bn(x1b, 2)]
    for h, y in enumerate(conv_bn(h2, 3)):
        o_ref[h] = jnp.maximum(x1[h] + y, 0.0)


def _prep_body(ws_ref, wb_ref, *, C):
    nblk = wb_ref.shape[1]
    W = nblk * _PB
    WC = W * C
    _, win, starts = _windows(W, C)
    pbc = _PB * C
    r = jax.lax.broadcasted_iota(jnp.int32, (WC, C), 0)
    c = jax.lax.broadcasted_iota(jnp.int32, (WC, C), 1)
    p1 = (r % C == c).astype(jnp.bfloat16)
    p2 = p1.T
    xo_blk = jax.lax.broadcasted_iota(jnp.int32, (WC, WC), 0) // C
    xi_blk = jax.lax.broadcasted_iota(jnp.int32, (WC, WC), 1) // C
    off = xi_blk - xo_blk + 1
    for band in range(6):
        acc = jnp.zeros((WC, WC), jnp.bfloat16)
        for kx in range(3):
            m = ws_ref[band, kx].astype(jnp.bfloat16)
            mp = jnp.dot(m, p2, preferred_element_type=jnp.float32)
            t = jnp.dot(p1, mp.astype(jnp.bfloat16),
                        preferred_element_type=jnp.float32)
            acc = jnp.where(off == kx, t.astype(jnp.bfloat16), acc)
        for w0 in range(nblk):
            wb_ref[band, w0] = acc[pbc * w0:pbc * (w0 + 1),
                                   starts[w0]:starts[w0] + win]


def kernel(x, b1_w1, b1_scale1, b1_bias1, b1_w2, b1_scale2, b1_bias2,
           b2_w1, b2_scale1, b2_bias1, b2_w2, b2_scale2, b2_bias2):
    N, H, W, C = x.shape
    WC = W * C
    NSPLIT = 2 if N % 256 == 0 else 1
    NL = N // NSPLIT
    nblk, win, _ = _windows(W, C)

    ws = jnp.stack([
        b1_w1 * b1_scale1, b1_w2 * b1_scale2,
        b2_w1 * b2_scale1, b2_w2 * b2_scale2,
    ]).reshape(12, 3, C, C).swapaxes(-1, -2)
    wbt = pl.pallas_call(
        functools.partial(_prep_body, C=C),
        out_shape=jax.ShapeDtypeStruct((12, nblk, _PB * C, win), jnp.bfloat16),
        grid=(2,),
        in_specs=[pl.BlockSpec((6, 3, C, C), lambda i: (i, 0, 0, 0))],
        out_specs=pl.BlockSpec((6, nblk, _PB * C, win), lambda i: (i, 0, 0, 0)),
        compiler_params=pltpu.CompilerParams(
            dimension_semantics=("parallel",),
        ),
    )(ws)

    bias = jnp.stack([
        jnp.tile(b1_bias1, W), jnp.tile(b1_bias2, W),
        jnp.tile(b2_bias1, W), jnp.tile(b2_bias2, W),
    ]).reshape(4, WC, 1).astype(jnp.float32)

    xt = x.transpose(1, 2, 3, 0).reshape(H, WC, N)

    out = pl.pallas_call(
        functools.partial(_body, H=H, C=C),
        out_shape=jax.ShapeDtypeStruct((H, WC, N), jnp.float32),
        grid=(NSPLIT,),
        in_specs=[
            pl.BlockSpec((H, WC, NL), lambda j: (0, 0, j)),
            pl.BlockSpec((12, nblk, _PB * C, win), lambda j: (0, 0, 0, 0)),
            pl.BlockSpec((4, WC, 1), lambda j: (0, 0, 0)),
        ],
        out_specs=pl.BlockSpec((H, WC, NL), lambda j: (0, 0, j)),
        compiler_params=pltpu.CompilerParams(
            dimension_semantics=("parallel",),
            vmem_limit_bytes=62 * 1024 * 1024,
        ),
    )(xt, wbt, bias)
    return out.reshape(H, W, C, N).transpose(3, 0, 1, 2)


# speedup vs baseline: 2.1139x; 1.1711x over previous
import functools

import jax
import jax.numpy as jnp
from jax.experimental import pallas as pl
from jax.experimental.pallas import tpu as pltpu

_PB = 4


def _windows(W, C):
    WC = W * C
    nblk = max(W // _PB, 1)
    win = min(_PB + 2, W) * C
    starts = [min(max(C * (_PB * w0 - 1), 0), WC - win) for w0 in range(nblk)]
    return nblk, win, starts


def _body(x_ref, ws_ref, b_ref, o_ref, *, H, C):
    f32, bf = jnp.float32, jnp.bfloat16
    WC = x_ref.shape[1]
    W = WC // C
    nblk, win, starts = _windows(W, C)
    pbc = _PB * C

    r = jax.lax.broadcasted_iota(jnp.int32, (pbc, C), 0)
    c = jax.lax.broadcasted_iota(jnp.int32, (pbc, C), 1)
    p1 = (r % C == c).astype(bf)
    q = jax.lax.broadcasted_iota(jnp.int32, (C, win), 1)
    cq = jax.lax.broadcasted_iota(jnp.int32, (C, win), 0)
    p2 = (q % C == cq).astype(bf)
    qmr = (jax.lax.broadcasted_iota(jnp.int32, (pbc, win), 1) // C
           - jax.lax.broadcasted_iota(jnp.int32, (pbc, win), 0) // C)
    dvs = [starts[w0] // C - _PB * w0 + 1 for w0 in range(nblk)]
    wblk = []
    for band in range(12):
        ts = []
        for kx in range(3):
            m = ws_ref[band, kx].astype(bf)
            mp = jnp.dot(m, p2, preferred_element_type=f32)
            ts.append(jnp.dot(p1, mp.astype(bf),
                              preferred_element_type=f32).astype(bf))
        by_dv = {}
        for dv in set(dvs):
            acc = jnp.zeros((pbc, win), bf)
            for kx in range(3):
                acc = jnp.where(qmr == kx - dv, ts[kx], acc)
            by_dv[dv] = acc
        wblk.append([by_dv[dv] for dv in dvs])

    def conv_bn(a, i):
        outs = []
        for h in range(H):
            blocks = []
            for w0 in range(nblk):
                s = starts[w0]
                y = jnp.dot(wblk[3 * i + 1][w0], a[h][s:s + win],
                            preferred_element_type=f32)
                if h > 0:
                    y += jnp.dot(wblk[3 * i][w0], a[h - 1][s:s + win],
                                 preferred_element_type=f32)
                if h < H - 1:
                    y += jnp.dot(wblk[3 * i + 2][w0], a[h + 1][s:s + win],
                                 preferred_element_type=f32)
                blocks.append(y + b_ref[i, pbc * w0:pbc * (w0 + 1)])
            outs.append(jnp.concatenate(blocks, axis=0)
                        if nblk > 1 else blocks[0])
        return outs

    x0 = [x_ref[h] for h in range(H)]
    x0b = [v.astype(bf) for v in x0]
    h1 = [jnp.maximum(y, 0.0).astype(bf) for y in conv_bn(x0b, 0)]
    x1 = [jnp.maximum(x0[h] + y, 0.0)
          for h, y in enumerate(conv_bn(h1, 1))]
    x1b = [v.astype(bf) for v in x1]
    h2 = [jnp.maximum(y, 0.0).astype(bf) for y in conv_bn(x1b, 2)]
    for h, y in enumerate(conv_bn(h2, 3)):
        o_ref[h] = jnp.maximum(x1[h] + y, 0.0)


def kernel(x, b1_w1, b1_scale1, b1_bias1, b1_w2, b1_scale2, b1_bias2,
           b2_w1, b2_scale1, b2_bias1, b2_w2, b2_scale2, b2_bias2):
    N, H, W, C = x.shape
    WC = W * C
    NSPLIT = 2 if N % 256 == 0 else 1
    NL = N // NSPLIT

    ws = jnp.stack([
        b1_w1 * b1_scale1, b1_w2 * b1_scale2,
        b2_w1 * b2_scale1, b2_w2 * b2_scale2,
    ]).reshape(12, 3, C, C).swapaxes(-1, -2)
    bias = jnp.stack([
        jnp.tile(b1_bias1, W), jnp.tile(b1_bias2, W),
        jnp.tile(b2_bias1, W), jnp.tile(b2_bias2, W),
    ]).reshape(4, WC, 1).astype(jnp.float32)

    xt = x.transpose(1, 2, 3, 0).reshape(H, WC, N)

    out = pl.pallas_call(
        functools.partial(_body, H=H, C=C),
        out_shape=jax.ShapeDtypeStruct((H, WC, N), jnp.float32),
        grid=(NSPLIT,),
        in_specs=[
            pl.BlockSpec((H, WC, NL), lambda j: (0, 0, j)),
            pl.BlockSpec((12, 3, C, C), lambda j: (0, 0, 0, 0)),
            pl.BlockSpec((4, WC, 1), lambda j: (0, 0, 0)),
        ],
        out_specs=pl.BlockSpec((H, WC, NL), lambda j: (0, 0, j)),
        compiler_params=pltpu.CompilerParams(
            dimension_semantics=("parallel",),
            vmem_limit_bytes=62 * 1024 * 1024,
        ),
    )(xt, ws, bias)
    return out.reshape(H, W, C, N).transpose(3, 0, 1, 2)
